# baseline (speedup 1.0000x reference)
import jax
import jax.numpy as jnp
from jax.experimental import pallas as pl
from jax.experimental.pallas import tpu as pltpu

K = 3
TAPS = 5
C1 = 16
C2 = 32
HID = 50
GPI = 128
GH = 64
G4 = 4 * GH
D1 = 32
OUT = 1


def _fused_kernel(xq_ref, xw_ref, bf_ref, c0_ref, c0b_ref, cl_ref, clb_ref,
                  whh_ref, fw1_ref, fb1_ref, fw2_ref, fb2_ref, out_ref):
    Lp4, BB, Cin = xq_ref.shape
    L = Lp4 - 4
    HB = BB // 2

    xq = xq_ref[...]

    xcat = jnp.concatenate(
        [xq[m:m + L].reshape(L * BB, Cin) for m in range(TAPS)], axis=1)
    xg = jnp.dot(xcat, xw_ref[...],
                 preferred_element_type=jnp.float32).astype(jnp.bfloat16)
    xg = (xg + bf_ref[...]).reshape(L, BB, G4)

    corr0 = jnp.dot(xq[2], c0_ref[...],
                    preferred_element_type=jnp.float32) + c0b_ref[...]
    corrL = jnp.dot(xq[L + 1], cl_ref[...],
                    preferred_element_type=jnp.float32) + clb_ref[...]

    whh = whh_ref[...]

    def gates(g):
        i = 0.5 * jnp.tanh(g[:, 0 * GH:1 * GH]) + 0.5
        f = 0.5 * jnp.tanh(g[:, 1 * GH:2 * GH]) + 0.5
        gg = jnp.tanh(g[:, 2 * GH:3 * GH])
        o = 0.5 * jnp.tanh(g[:, 3 * GH:4 * GH]) + 0.5
        return i, f, gg, o

    def first_step(g0):
        i, _, gg, o = gates(g0.astype(jnp.float32))
        c = i * gg
        h = (o * jnp.tanh(c)).astype(jnp.bfloat16)
        return h, c

    def step(xg_t, h, c):
        g = xg_t + jnp.dot(h, whh, preferred_element_type=jnp.float32)
        i, f, gg, o = gates(g)
        c = f * c + i * gg
        h = (o * jnp.tanh(c)).astype(jnp.bfloat16)
        return h, c

    h_a, c_a = first_step(xg[0, :HB] - corr0[:HB])
    h_b, c_b = first_step(xg[0, HB:] - corr0[HB:])
    for t in range(1, L):
        xg_t = xg[t] if t < L - 1 else xg[L - 1] - corrL
        h_a, c_a = step(xg_t[:HB], h_a, c_a)
        h_b, c_b = step(xg_t[HB:], h_b, c_b)
    h = jnp.concatenate([h_a, h_b], axis=0)

    d1 = jnp.dot(h, fw1_ref[...], preferred_element_type=jnp.float32) + fb1_ref[...]
    out = jnp.sum(d1 * fw2_ref[...], axis=-1, keepdims=True) + fb2_ref[...]
    out_ref[...] = out.astype(out_ref.dtype)


def _regate(w):
    out = jnp.zeros((w.shape[0], G4), w.dtype)
    for k in range(4):
        out = out.at[:, k * GH:k * GH + HID].set(w[:, k * GPI:k * GPI + HID])
    return out


def kernel(x, cw1, cb1, cw2, cb2, w_ih, w_hh, b_l, fw1, fb1, fw2, fb2):
    B, L, Cin = x.shape
    bp8 = ((max(B, 8) + 7) // 8) * 8
    block_b = min(bp8, 512)
    BP = ((bp8 + block_b - 1) // block_b) * block_b
    nblk = BP // block_b

    x_t = jnp.transpose(x, (1, 0, 2))
    x_t = jnp.pad(x_t, ((2, 2), (0, BP - B), (0, 0))).astype(jnp.bfloat16)

    wih64 = _regate(w_ih)
    s = jnp.concatenate([jnp.full((2 * GH,), 0.5), jnp.ones((GH,)),
                         jnp.full((GH,), 0.5)]).astype(jnp.float32)
    wih_s = wih64 * s
    xw = jnp.zeros((TAPS, Cin, G4), jnp.float32)
    for i in range(K):
        for j in range(K):
            xw = xw.at[i + j].add(cw1[i] @ (cw2[j] @ wih_s))
    xw_cat = xw.reshape(TAPS * Cin, G4).astype(jnp.bfloat16)
    bfold = ((cb1 @ (cw2[0] + cw2[1] + cw2[2]) + cb2) @ wih_s
             + _regate(b_l) * s).astype(jnp.bfloat16)
    c0 = (cw1[2] @ (cw2[0] @ wih_s)).astype(jnp.bfloat16)
    c0b = ((cb1 @ cw2[0]) @ wih_s).astype(jnp.bfloat16)
    cl = (cw1[0] @ (cw2[2] @ wih_s)).astype(jnp.bfloat16)
    clb = ((cb1 @ cw2[2]) @ wih_s).astype(jnp.bfloat16)

    whh_s = (_regate(w_hh[:GH]) * s).astype(jnp.bfloat16)
    fw1_g = fw1[:GH].astype(jnp.bfloat16)

    out = pl.pallas_call(
        _fused_kernel,
        out_shape=jax.ShapeDtypeStruct((BP, OUT), jnp.float32),
        grid=(nblk,),
        in_specs=[
            pl.BlockSpec((L + 4, block_b, Cin), lambda b: (0, b, 0)),
            pl.BlockSpec((TAPS * Cin, G4), lambda b: (0, 0)),
            pl.BlockSpec((1, G4), lambda b: (0, 0)),
            pl.BlockSpec((Cin, G4), lambda b: (0, 0)),
            pl.BlockSpec((1, G4), lambda b: (0, 0)),
            pl.BlockSpec((Cin, G4), lambda b: (0, 0)),
            pl.BlockSpec((1, G4), lambda b: (0, 0)),
            pl.BlockSpec((GH, G4), lambda b: (0, 0)),
            pl.BlockSpec((GH, D1), lambda b: (0, 0)),
            pl.BlockSpec((1, D1), lambda b: (0, 0)),
            pl.BlockSpec((1, D1), lambda b: (0, 0)),
            pl.BlockSpec((1, OUT), lambda b: (0, 0)),
        ],
        out_specs=pl.BlockSpec((block_b, OUT), lambda b: (b, 0)),
        compiler_params=pltpu.CompilerParams(
            dimension_semantics=("arbitrary",)),
    )(x_t, xw_cat, bfold, c0, c0b, cl, clb,
      whh_s, fw1_g, fb1, fw2, fb2)
    return out[:B]


# speedup vs baseline: 1.0691x; 1.0691x over previous
import jax
import jax.numpy as jnp
from jax.experimental import pallas as pl
from jax.experimental.pallas import tpu as pltpu

K = 3
TAPS = 5
C1 = 16
C2 = 32
HID = 50
GPI = 128
GH = 64
G4 = 4 * GH
D1 = 32
OUT = 1


def _fused_kernel(xq_ref, xw_ref, bf_ref, c0_ref, c0b_ref, cl_ref, clb_ref,
                  whh_ref, fw1_ref, fb1_ref, fw2_ref, fb2_ref, out_ref):
    Lp4, BB, Cin = xq_ref.shape
    L = Lp4 - 4
    HB = BB // 2

    xq = xq_ref[...]

    NCH = 4
    LC = L // NCH
    xw = xw_ref[...]
    bf = bf_ref[...]
    xg_chunks = []
    for ch in range(NCH):
        t0 = ch * LC
        xcat = jnp.concatenate(
            [xq[t0 + m:t0 + m + LC].reshape(LC * BB, Cin)
             for m in range(TAPS)], axis=1)
        xg_ch = jnp.dot(xcat, xw,
                        preferred_element_type=jnp.float32).astype(jnp.bfloat16)
        xg_chunks.append((xg_ch + bf).reshape(LC, BB, G4))

    def xg_at(t):
        return xg_chunks[t // LC][t % LC]

    corr0 = jnp.dot(xq[2], c0_ref[...],
                    preferred_element_type=jnp.float32) + c0b_ref[...]
    corrL = jnp.dot(xq[L + 1], cl_ref[...],
                    preferred_element_type=jnp.float32) + clb_ref[...]

    whh = whh_ref[...]

    def gates(g):
        i = 0.5 * jnp.tanh(g[:, 0 * GH:1 * GH]) + 0.5
        f = 0.5 * jnp.tanh(g[:, 1 * GH:2 * GH]) + 0.5
        gg = jnp.tanh(g[:, 2 * GH:3 * GH])
        o = 0.5 * jnp.tanh(g[:, 3 * GH:4 * GH]) + 0.5
        return i, f, gg, o

    def first_step(g0):
        i, _, gg, o = gates(g0.astype(jnp.float32))
        c = i * gg
        h = (o * jnp.tanh(c)).astype(jnp.bfloat16)
        return h, c

    def step(xg_t, h, c):
        g = xg_t + jnp.dot(h, whh, preferred_element_type=jnp.float32)
        i, f, gg, o = gates(g)
        c = f * c + i * gg
        h = (o * jnp.tanh(c)).astype(jnp.bfloat16)
        return h, c

    g0 = xg_at(0) - corr0
    h_a, c_a = first_step(g0[:HB])
    h_b, c_b = first_step(g0[HB:])
    for t in range(1, L):
        xg_t = xg_at(t) if t < L - 1 else xg_at(L - 1) - corrL
        h_a, c_a = step(xg_t[:HB], h_a, c_a)
        h_b, c_b = step(xg_t[HB:], h_b, c_b)
    h = jnp.concatenate([h_a, h_b], axis=0)

    d1 = jnp.dot(h, fw1_ref[...], preferred_element_type=jnp.float32) + fb1_ref[...]
    out = jnp.sum(d1 * fw2_ref[...], axis=-1, keepdims=True) + fb2_ref[...]
    out_ref[...] = out.astype(out_ref.dtype)


def _regate(w):
    out = jnp.zeros((w.shape[0], G4), w.dtype)
    for k in range(4):
        out = out.at[:, k * GH:k * GH + HID].set(w[:, k * GPI:k * GPI + HID])
    return out


def kernel(x, cw1, cb1, cw2, cb2, w_ih, w_hh, b_l, fw1, fb1, fw2, fb2):
    B, L, Cin = x.shape
    bp8 = ((max(B, 8) + 7) // 8) * 8
    block_b = min(bp8, 512)
    BP = ((bp8 + block_b - 1) // block_b) * block_b
    nblk = BP // block_b

    x_t = jnp.transpose(x, (1, 0, 2))
    x_t = jnp.pad(x_t, ((2, 2), (0, BP - B), (0, 0))).astype(jnp.bfloat16)

    wih64 = _regate(w_ih)
    s = jnp.concatenate([jnp.full((2 * GH,), 0.5), jnp.ones((GH,)),
                         jnp.full((GH,), 0.5)]).astype(jnp.float32)
    wih_s = wih64 * s
    xw = jnp.zeros((TAPS, Cin, G4), jnp.float32)
    for i in range(K):
        for j in range(K):
            xw = xw.at[i + j].add(cw1[i] @ (cw2[j] @ wih_s))
    xw_cat = xw.reshape(TAPS * Cin, G4).astype(jnp.bfloat16)
    bfold = ((cb1 @ (cw2[0] + cw2[1] + cw2[2]) + cb2) @ wih_s
             + _regate(b_l) * s).astype(jnp.bfloat16)
    c0 = (cw1[2] @ (cw2[0] @ wih_s)).astype(jnp.bfloat16)
    c0b = ((cb1 @ cw2[0]) @ wih_s).astype(jnp.bfloat16)
    cl = (cw1[0] @ (cw2[2] @ wih_s)).astype(jnp.bfloat16)
    clb = ((cb1 @ cw2[2]) @ wih_s).astype(jnp.bfloat16)

    whh_s = (_regate(w_hh[:GH]) * s).astype(jnp.bfloat16)
    fw1_g = fw1[:GH].astype(jnp.bfloat16)

    out = pl.pallas_call(
        _fused_kernel,
        out_shape=jax.ShapeDtypeStruct((BP, OUT), jnp.float32),
        grid=(nblk,),
        in_specs=[
            pl.BlockSpec((L + 4, block_b, Cin), lambda b: (0, b, 0)),
            pl.BlockSpec((TAPS * Cin, G4), lambda b: (0, 0)),
            pl.BlockSpec((1, G4), lambda b: (0, 0)),
            pl.BlockSpec((Cin, G4), lambda b: (0, 0)),
            pl.BlockSpec((1, G4), lambda b: (0, 0)),
            pl.BlockSpec((Cin, G4), lambda b: (0, 0)),
            pl.BlockSpec((1, G4), lambda b: (0, 0)),
            pl.BlockSpec((GH, G4), lambda b: (0, 0)),
            pl.BlockSpec((GH, D1), lambda b: (0, 0)),
            pl.BlockSpec((1, D1), lambda b: (0, 0)),
            pl.BlockSpec((1, D1), lambda b: (0, 0)),
            pl.BlockSpec((1, OUT), lambda b: (0, 0)),
        ],
        out_specs=pl.BlockSpec((block_b, OUT), lambda b: (b, 0)),
        compiler_params=pltpu.CompilerParams(
            dimension_semantics=("arbitrary",)),
    )(x_t, xw_cat, bfold, c0, c0b, cl, clb,
      whh_s, fw1_g, fb1, fw2, fb2)
    return out[:B]
